# baseline (speedup 1.0000x reference)
import functools

import jax
import jax.numpy as jnp
from jax import lax
from jax.experimental import pallas as pl
from jax.experimental.pallas import tpu as pltpu
from jax.experimental.pallas import tpu_sc as plsc

N_NODES = 10000
N_PAD = 10240
N_EDGES = 320000
D = 128
NW = 32
E_PER_W = N_EDGES // NW
B = 80
NB = E_PER_W // B
IC = 25
ROWS_PER_TILE = N_PAD // 16

EB = 2560
NEB = N_EDGES // EB

_mesh = plsc.VectorSubcoreMesh(core_axis_name="c", subcore_axis_name="s")


@functools.partial(
    pl.kernel,
    mesh=_mesh,
    out_type=jax.ShapeDtypeStruct((2, N_PAD, D), jnp.float32),
    scratch_types=[
        pltpu.VMEM((IC, B), jnp.int32),
        pltpu.VMEM((IC, B), jnp.int32),
        pltpu.VMEM((B, D), jnp.float32),
        pltpu.VMEM((B, D), jnp.float32),
        pltpu.VMEM_SHARED((N_PAD, D), jnp.float32),
        pltpu.SemaphoreType.DMA,
        pltpu.SemaphoreType.DMA,
    ],
)
def _agg_kernel(hs_hbm, src_hbm, dst_hbm, zeros_hbm, out_hbm,
                src_v, dst_v, rows0, rows1, acc, sem0, sem1):
    cid = lax.axis_index("c")
    sid = lax.axis_index("s")
    wid = cid * 16 + sid
    pltpu.sync_copy(zeros_hbm, acc.at[pl.ds(sid * ROWS_PER_TILE, ROWS_PER_TILE)])
    plsc.subcore_barrier()

    def chunk(ch, carry):
        pltpu.sync_copy(src_hbm.at[wid, ch], src_v)
        pltpu.sync_copy(dst_hbm.at[wid, ch], dst_v)
        pltpu.async_copy(hs_hbm.at[src_v.at[0]], rows0, sem0)

        def body(i, c2):
            pltpu.async_copy(hs_hbm.at[src_v.at[2 * i + 1]], rows1, sem1)
            pltpu.make_async_copy(hs_hbm.at[src_v.at[2 * i]], rows0, sem0).wait()
            pltpu.sync_copy(rows0, acc.at[dst_v.at[2 * i]], add=True)
            pltpu.async_copy(hs_hbm.at[src_v.at[2 * i + 2]], rows0, sem0)
            pltpu.make_async_copy(hs_hbm.at[src_v.at[2 * i + 1]], rows1, sem1).wait()
            pltpu.sync_copy(rows1, acc.at[dst_v.at[2 * i + 1]], add=True)
            return c2

        lax.fori_loop(0, (IC - 1) // 2, body, 0)
        pltpu.make_async_copy(hs_hbm.at[src_v.at[IC - 1]], rows0, sem0).wait()
        pltpu.sync_copy(rows0, acc.at[dst_v.at[IC - 1]], add=True)
        return carry

    lax.fori_loop(0, NB // IC, chunk, 0)
    plsc.subcore_barrier()
    pltpu.sync_copy(
        acc.at[pl.ds(sid * ROWS_PER_TILE, ROWS_PER_TILE)],
        out_hbm.at[cid, pl.ds(sid * ROWS_PER_TILE, ROWS_PER_TILE)],
    )


def _deg_body(dst_ref, c_ref):
    i = pl.program_id(0)

    @pl.when(i == 0)
    def _():
        c_ref[...] = jnp.zeros_like(c_ref)

    d = dst_ref[0]
    q = lax.shift_right_logical(d, 7)
    r = lax.bitwise_and(d, 127)
    k = lax.broadcasted_iota(jnp.int32, (EB, 128), 1)
    m2 = (q == k).astype(jnp.bfloat16)
    m1 = (r == k).astype(jnp.bfloat16)
    c_ref[...] += lax.dot_general(
        m2, m1, (((0,), (0,)), ((), ())),
        preferred_element_type=jnp.float32)


def _deg_tc(dst3):
    return pl.pallas_call(
        _deg_body,
        grid=(NEB,),
        in_specs=[pl.BlockSpec((1, EB, 1), lambda i: (i, 0, 0))],
        out_specs=pl.BlockSpec((128, 128), lambda i: (0, 0)),
        out_shape=jax.ShapeDtypeStruct((128, 128), jnp.float32),
    )(dst3)


def _pick_s(cnt_ref, blk, i):
    n = lax.broadcasted_iota(jnp.int32, (blk, 1), 0) + i * blk
    q = lax.shift_right_logical(n, 7)
    r = lax.bitwise_and(n, 127)
    lane = lax.broadcasted_iota(jnp.int32, (blk, 128), 1)
    eq = (q == lane).astype(jnp.bfloat16)
    picked = lax.dot_general(
        eq, cnt_ref[...].astype(jnp.bfloat16), (((1,), (0,)), ((), ())),
        preferred_element_type=jnp.float32)
    cnt = jnp.sum(jnp.where(r == lane, picked, 0.0), axis=1, keepdims=True)
    return lax.rsqrt(cnt + 1.0)


def _lin_body(x_ref, w_ref, cnt_ref, hs_ref, s_ref):
    blk = x_ref.shape[0]
    s2 = _pick_s(cnt_ref, blk, pl.program_id(0))
    h = jnp.dot(x_ref[...], w_ref[...], preferred_element_type=jnp.float32)
    hs_ref[...] = h * s2
    s_ref[...] = s2


def _lin_tc(x, W, degc):
    blk = 1000
    return pl.pallas_call(
        _lin_body,
        grid=(N_NODES // blk,),
        in_specs=[
            pl.BlockSpec((blk, D), lambda i: (i, 0)),
            pl.BlockSpec((D, D), lambda i: (0, 0)),
            pl.BlockSpec((128, 128), lambda i: (0, 0)),
        ],
        out_specs=[
            pl.BlockSpec((blk, D), lambda i: (i, 0)),
            pl.BlockSpec((blk, 1), lambda i: (i, 0)),
        ],
        out_shape=[
            jax.ShapeDtypeStruct((N_NODES, D), jnp.float32),
            jax.ShapeDtypeStruct((N_NODES, 1), jnp.float32),
        ],
    )(x, W, degc)


def _fin_body(p_ref, hs_ref, s_ref, b_ref, out_ref):
    tot = p_ref[0] + p_ref[1] + hs_ref[...]
    out_ref[...] = jnp.maximum(s_ref[...] * tot + b_ref[...], 0.0)


def _fin_tc(parts, hs, s2d, b2):
    blk = 1000
    return pl.pallas_call(
        _fin_body,
        grid=(N_NODES // blk,),
        in_specs=[
            pl.BlockSpec((2, blk, D), lambda i: (0, i, 0)),
            pl.BlockSpec((blk, D), lambda i: (i, 0)),
            pl.BlockSpec((blk, 1), lambda i: (i, 0)),
            pl.BlockSpec((1, D), lambda i: (0, 0)),
        ],
        out_specs=pl.BlockSpec((blk, D), lambda i: (i, 0)),
        out_shape=jax.ShapeDtypeStruct((N_NODES, D), jnp.float32),
    )(parts, hs, s2d, b2)


def kernel(x, edge_index, W, b):
    ei = edge_index.astype(jnp.int32)
    src4 = ei[0].reshape(NW, NB // IC, IC, B)
    dst4 = ei[1].reshape(NW, NB // IC, IC, B)
    dstb = ei[1].reshape(NEB, EB, 1)
    zeros_row = jnp.zeros((ROWS_PER_TILE, D), jnp.float32)

    degc = _deg_tc(dstb)
    hs, s2d = _lin_tc(x, W, degc)
    parts = _agg_kernel(hs, src4, dst4, zeros_row)
    return _fin_tc(parts, hs, s2d, b[None, :])


# speedup vs baseline: 2.1494x; 2.1494x over previous
import functools

import jax
import jax.numpy as jnp
from jax import lax
from jax.experimental import pallas as pl
from jax.experimental.pallas import tpu as pltpu
from jax.experimental.pallas import tpu_sc as plsc

N_NODES = 10000
N_PAD = 10240
N_EDGES = 320000
D = 128
NW = 32
E_PER_W = N_EDGES // NW
B = 80
NB = E_PER_W // B
IC = 25
ROWS_PER_TILE = N_PAD // 16

EB = 2560
NEB = N_EDGES // EB

_mesh = plsc.VectorSubcoreMesh(core_axis_name="c", subcore_axis_name="s")


@functools.partial(
    pl.kernel,
    mesh=_mesh,
    out_type=jax.ShapeDtypeStruct((2, N_PAD, D), jnp.float32),
    scratch_types=[
        pltpu.VMEM((IC, B), jnp.int32),
        pltpu.VMEM((IC, B), jnp.int32),
        pltpu.VMEM((B, D), jnp.float32),
        pltpu.VMEM((B, D), jnp.float32),
        pltpu.VMEM_SHARED((N_PAD, D), jnp.float32),
        pltpu.SemaphoreType.DMA,
        pltpu.SemaphoreType.DMA,
    ],
)
def _agg_kernel(hs_hbm, src_hbm, dst_hbm, zeros_hbm, out_hbm,
                src_v, dst_v, rows0, rows1, acc, sem0, sem1):
    cid = lax.axis_index("c")
    sid = lax.axis_index("s")
    wid = cid * 16 + sid
    pltpu.sync_copy(zeros_hbm, acc.at[pl.ds(sid * ROWS_PER_TILE, ROWS_PER_TILE)])
    plsc.subcore_barrier()

    def chunk(ch, carry):
        pltpu.sync_copy(src_hbm.at[wid, ch], src_v)
        pltpu.sync_copy(dst_hbm.at[wid, ch], dst_v)
        pltpu.async_copy(hs_hbm.at[src_v.at[0]], rows0, sem0)

        def body(i, c2):
            pltpu.async_copy(hs_hbm.at[src_v.at[2 * i + 1]], rows1, sem1)
            pltpu.make_async_copy(hs_hbm.at[src_v.at[2 * i]], rows0, sem0).wait()
            pltpu.sync_copy(rows0, acc.at[dst_v.at[2 * i]], add=True)
            pltpu.async_copy(hs_hbm.at[src_v.at[2 * i + 2]], rows0, sem0)
            pltpu.make_async_copy(hs_hbm.at[src_v.at[2 * i + 1]], rows1, sem1).wait()
            pltpu.sync_copy(rows1, acc.at[dst_v.at[2 * i + 1]], add=True)
            return c2

        lax.fori_loop(0, (IC - 1) // 2, body, 0)
        pltpu.make_async_copy(hs_hbm.at[src_v.at[IC - 1]], rows0, sem0).wait()
        pltpu.sync_copy(rows0, acc.at[dst_v.at[IC - 1]], add=True)
        return carry

    lax.fori_loop(0, NB // IC, chunk, 0)
    plsc.subcore_barrier()
    pltpu.sync_copy(
        acc.at[pl.ds(sid * ROWS_PER_TILE, ROWS_PER_TILE)],
        out_hbm.at[cid, pl.ds(sid * ROWS_PER_TILE, ROWS_PER_TILE)],
    )


def _deg_body(dst_ref, c_ref):
    i = pl.program_id(0)

    @pl.when(i == 0)
    def _():
        c_ref[...] = jnp.zeros_like(c_ref)

    d = dst_ref[0]
    q = lax.shift_right_logical(d, 7)
    r = lax.bitwise_and(d, 127)
    k = lax.broadcasted_iota(jnp.int32, (128, EB), 0)
    m2t = (q == k).astype(jnp.bfloat16)
    m1t = (r == k).astype(jnp.bfloat16)
    c_ref[...] += lax.dot_general(
        m2t, m1t, (((1,), (1,)), ((), ())),
        preferred_element_type=jnp.float32)


def _deg_tc(dst3):
    return pl.pallas_call(
        _deg_body,
        grid=(NEB,),
        in_specs=[pl.BlockSpec((1, 1, EB), lambda i: (i, 0, 0))],
        out_specs=pl.BlockSpec((128, 128), lambda i: (0, 0)),
        out_shape=jax.ShapeDtypeStruct((128, 128), jnp.float32),
    )(dst3)


def _pick_s(cnt_ref, blk, i):
    n = lax.broadcasted_iota(jnp.int32, (blk, 1), 0) + i * blk
    q = lax.shift_right_logical(n, 7)
    r = lax.bitwise_and(n, 127)
    lane = lax.broadcasted_iota(jnp.int32, (blk, 128), 1)
    eq = (q == lane).astype(jnp.bfloat16)
    picked = lax.dot_general(
        eq, cnt_ref[...].astype(jnp.bfloat16), (((1,), (0,)), ((), ())),
        preferred_element_type=jnp.float32)
    cnt = jnp.sum(jnp.where(r == lane, picked, 0.0), axis=1, keepdims=True)
    return lax.rsqrt(cnt + 1.0)


def _lin_body(x_ref, w_ref, cnt_ref, hs_ref, s_ref):
    blk = x_ref.shape[0]
    s2 = _pick_s(cnt_ref, blk, pl.program_id(0))
    h = jnp.dot(x_ref[...], w_ref[...], preferred_element_type=jnp.float32)
    hs_ref[...] = h * s2
    s_ref[...] = s2


def _lin_tc(x, W, degc):
    blk = 1000
    return pl.pallas_call(
        _lin_body,
        grid=(N_NODES // blk,),
        in_specs=[
            pl.BlockSpec((blk, D), lambda i: (i, 0)),
            pl.BlockSpec((D, D), lambda i: (0, 0)),
            pl.BlockSpec((128, 128), lambda i: (0, 0)),
        ],
        out_specs=[
            pl.BlockSpec((blk, D), lambda i: (i, 0)),
            pl.BlockSpec((blk, 1), lambda i: (i, 0)),
        ],
        out_shape=[
            jax.ShapeDtypeStruct((N_NODES, D), jnp.float32),
            jax.ShapeDtypeStruct((N_NODES, 1), jnp.float32),
        ],
    )(x, W, degc)


def _fin_body(p_ref, hs_ref, s_ref, b_ref, out_ref):
    tot = p_ref[0] + p_ref[1] + hs_ref[...]
    out_ref[...] = jnp.maximum(s_ref[...] * tot + b_ref[...], 0.0)


def _fin_tc(parts, hs, s2d, b2):
    blk = 1000
    return pl.pallas_call(
        _fin_body,
        grid=(N_NODES // blk,),
        in_specs=[
            pl.BlockSpec((2, blk, D), lambda i: (0, i, 0)),
            pl.BlockSpec((blk, D), lambda i: (i, 0)),
            pl.BlockSpec((blk, 1), lambda i: (i, 0)),
            pl.BlockSpec((1, D), lambda i: (0, 0)),
        ],
        out_specs=pl.BlockSpec((blk, D), lambda i: (i, 0)),
        out_shape=jax.ShapeDtypeStruct((N_NODES, D), jnp.float32),
    )(parts, hs, s2d, b2)


def kernel(x, edge_index, W, b):
    ei = edge_index.astype(jnp.int32)
    src4 = ei[0].reshape(NW, NB // IC, IC, B)
    dst4 = ei[1].reshape(NW, NB // IC, IC, B)
    dstb = ei[1].reshape(NEB, 1, EB)
    zeros_row = jnp.zeros((ROWS_PER_TILE, D), jnp.float32)

    degc = _deg_tc(dstb)
    hs, s2d = _lin_tc(x, W, degc)
    parts = _agg_kernel(hs, src4, dst4, zeros_row)
    return _fin_tc(parts, hs, s2d, b[None, :])


# speedup vs baseline: 2.7335x; 1.2718x over previous
import functools

import jax
import jax.numpy as jnp
from jax import lax
from jax.experimental import pallas as pl
from jax.experimental.pallas import tpu as pltpu
from jax.experimental.pallas import tpu_sc as plsc

N_NODES = 10000
N_PAD = 10240
N_EDGES = 320000
D = 128
NW = 32
E_PER_W = N_EDGES // NW
B = 80
NB = E_PER_W // B
IC = 25
ROWS_PER_TILE = N_PAD // 16

EB = 12800
NEB = N_EDGES // EB
NQ = 80

_mesh = plsc.VectorSubcoreMesh(core_axis_name="c", subcore_axis_name="s")


@functools.partial(
    pl.kernel,
    mesh=_mesh,
    out_type=jax.ShapeDtypeStruct((2, N_PAD, D), jnp.float32),
    scratch_types=[
        pltpu.VMEM((IC, B), jnp.int32),
        pltpu.VMEM((IC, B), jnp.int32),
        pltpu.VMEM((B, D), jnp.float32),
        pltpu.VMEM((B, D), jnp.float32),
        pltpu.VMEM((B, D), jnp.float32),
        pltpu.VMEM_SHARED((N_PAD, D), jnp.float32),
        pltpu.SemaphoreType.DMA,
        pltpu.SemaphoreType.DMA,
        pltpu.SemaphoreType.DMA,
    ],
)
def _agg_kernel(hs_hbm, src_hbm, dst_hbm, zeros_hbm, out_hbm,
                src_v, dst_v, rows0, rows1, rows2, acc, sem0, sem1, sem2):
    cid = lax.axis_index("c")
    sid = lax.axis_index("s")
    wid = cid * 16 + sid
    pltpu.sync_copy(zeros_hbm, acc.at[pl.ds(sid * ROWS_PER_TILE, ROWS_PER_TILE)])
    plsc.subcore_barrier()

    rows = (rows0, rows1, rows2)
    sems = (sem0, sem1, sem2)

    def chunk(ch, carry):
        pltpu.sync_copy(src_hbm.at[wid, ch], src_v)
        pltpu.sync_copy(dst_hbm.at[wid, ch], dst_v)
        pltpu.async_copy(hs_hbm.at[src_v.at[0]], rows0, sem0)
        pltpu.async_copy(hs_hbm.at[src_v.at[1]], rows1, sem1)

        def body(j, c2):
            for k in range(3):
                b = 3 * j + k
                kn = (k + 2) % 3
                pltpu.async_copy(hs_hbm.at[src_v.at[b + 2]], rows[kn], sems[kn])
                pltpu.make_async_copy(
                    hs_hbm.at[src_v.at[b]], rows[k], sems[k]).wait()
                pltpu.sync_copy(rows[k], acc.at[dst_v.at[b]], add=True)
            return c2

        lax.fori_loop(0, (IC - 2) // 3, body, 0)
        for b in range(3 * ((IC - 2) // 3), IC):
            k = b % 3
            if b + 2 < IC:
                kn = (b + 2) % 3
                pltpu.async_copy(hs_hbm.at[src_v.at[b + 2]], rows[kn], sems[kn])
            pltpu.make_async_copy(
                hs_hbm.at[src_v.at[b]], rows[k], sems[k]).wait()
            pltpu.sync_copy(rows[k], acc.at[dst_v.at[b]], add=True)
        return carry

    lax.fori_loop(0, NB // IC, chunk, 0)
    plsc.subcore_barrier()
    pltpu.sync_copy(
        acc.at[pl.ds(sid * ROWS_PER_TILE, ROWS_PER_TILE)],
        out_hbm.at[cid, pl.ds(sid * ROWS_PER_TILE, ROWS_PER_TILE)],
    )


def _deg_body(dst_ref, c_ref):
    i = pl.program_id(0)

    @pl.when(i == 0)
    def _():
        c_ref[...] = jnp.zeros_like(c_ref)

    d = dst_ref[0]
    q = lax.shift_right_logical(d, 7)
    r = lax.bitwise_and(d, 127)
    kq = lax.broadcasted_iota(jnp.int32, (NQ, EB), 0)
    k = lax.broadcasted_iota(jnp.int32, (128, EB), 0)
    m2t = (q == kq).astype(jnp.bfloat16)
    m1t = (r == k).astype(jnp.bfloat16)
    c_ref[...] += lax.dot_general(
        m2t, m1t, (((1,), (1,)), ((), ())),
        preferred_element_type=jnp.float32)


def _deg_tc(dst3):
    return pl.pallas_call(
        _deg_body,
        grid=(NEB,),
        in_specs=[pl.BlockSpec((1, 1, EB), lambda i: (i, 0, 0))],
        out_specs=pl.BlockSpec((NQ, 128), lambda i: (0, 0)),
        out_shape=jax.ShapeDtypeStruct((NQ, 128), jnp.float32),
    )(dst3)


def _pick_s(cnt_ref, blk, i):
    n = lax.broadcasted_iota(jnp.int32, (blk, 1), 0) + i * blk
    q = lax.shift_right_logical(n, 7)
    r = lax.bitwise_and(n, 127)
    lane = lax.broadcasted_iota(jnp.int32, (blk, 128), 1)
    laneq = lax.broadcasted_iota(jnp.int32, (blk, NQ), 1)
    eq = (q == laneq).astype(jnp.bfloat16)
    picked = lax.dot_general(
        eq, cnt_ref[...].astype(jnp.bfloat16), (((1,), (0,)), ((), ())),
        preferred_element_type=jnp.float32)
    cnt = jnp.sum(jnp.where(r == lane, picked, 0.0), axis=1, keepdims=True)
    return lax.rsqrt(cnt + 1.0)


def _lin_body(x_ref, w_ref, cnt_ref, hs_ref, s_ref):
    blk = x_ref.shape[0]
    s2 = _pick_s(cnt_ref, blk, pl.program_id(0))
    h = jnp.dot(x_ref[...], w_ref[...], preferred_element_type=jnp.float32)
    hs_ref[...] = h * s2
    s_ref[...] = s2


def _lin_tc(x, W, degc):
    blk = 1000
    return pl.pallas_call(
        _lin_body,
        grid=(N_NODES // blk,),
        in_specs=[
            pl.BlockSpec((blk, D), lambda i: (i, 0)),
            pl.BlockSpec((D, D), lambda i: (0, 0)),
            pl.BlockSpec((NQ, 128), lambda i: (0, 0)),
        ],
        out_specs=[
            pl.BlockSpec((blk, D), lambda i: (i, 0)),
            pl.BlockSpec((blk, 1), lambda i: (i, 0)),
        ],
        out_shape=[
            jax.ShapeDtypeStruct((N_NODES, D), jnp.float32),
            jax.ShapeDtypeStruct((N_NODES, 1), jnp.float32),
        ],
    )(x, W, degc)


def _fin_body(p_ref, hs_ref, s_ref, b_ref, out_ref):
    tot = p_ref[0] + p_ref[1] + hs_ref[...]
    out_ref[...] = jnp.maximum(s_ref[...] * tot + b_ref[...], 0.0)


def _fin_tc(parts, hs, s2d, b2):
    blk = 1000
    return pl.pallas_call(
        _fin_body,
        grid=(N_NODES // blk,),
        in_specs=[
            pl.BlockSpec((2, blk, D), lambda i: (0, i, 0)),
            pl.BlockSpec((blk, D), lambda i: (i, 0)),
            pl.BlockSpec((blk, 1), lambda i: (i, 0)),
            pl.BlockSpec((1, D), lambda i: (0, 0)),
        ],
        out_specs=pl.BlockSpec((blk, D), lambda i: (i, 0)),
        out_shape=jax.ShapeDtypeStruct((N_NODES, D), jnp.float32),
    )(parts, hs, s2d, b2)


def kernel(x, edge_index, W, b):
    ei = edge_index.astype(jnp.int32)
    src4 = ei[0].reshape(NW, NB // IC, IC, B)
    dst4 = ei[1].reshape(NW, NB // IC, IC, B)
    dstb = ei[1].reshape(NEB, 1, EB)
    zeros_row = jnp.zeros((ROWS_PER_TILE, D), jnp.float32)

    degc = _deg_tc(dstb)
    hs, s2d = _lin_tc(x, W, degc)
    parts = _agg_kernel(hs, src4, dst4, zeros_row)
    return _fin_tc(parts, hs, s2d, b[None, :])


# speedup vs baseline: 2.8966x; 1.0597x over previous
import functools

import jax
import jax.numpy as jnp
from jax import lax
from jax.experimental import pallas as pl
from jax.experimental.pallas import tpu as pltpu
from jax.experimental.pallas import tpu_sc as plsc

N_NODES = 10000
N_PAD = 10240
N_EDGES = 320000
D = 128
NW = 32
E_PER_W = N_EDGES // NW
B = 80
NB = E_PER_W // B
IC = 25
ROWS_PER_TILE = N_PAD // 16

EB = 12800
NEB = N_EDGES // EB
NQ = 80

_mesh = plsc.VectorSubcoreMesh(core_axis_name="c", subcore_axis_name="s")


@functools.partial(
    pl.kernel,
    mesh=_mesh,
    out_type=jax.ShapeDtypeStruct((2, N_PAD, D), jnp.float32),
    scratch_types=[
        pltpu.VMEM((IC, B), jnp.int32),
        pltpu.VMEM((IC, B), jnp.int32),
        pltpu.VMEM((B, D), jnp.float32),
        pltpu.VMEM((B, D), jnp.float32),
        pltpu.VMEM((B, D), jnp.float32),
        pltpu.VMEM_SHARED((N_PAD, D), jnp.float32),
        pltpu.SemaphoreType.DMA,
        pltpu.SemaphoreType.DMA,
        pltpu.SemaphoreType.DMA,
    ],
)
def _agg_kernel(hs_hbm, ei_hbm, zeros_hbm, out_hbm,
                src_v, dst_v, rows0, rows1, rows2, acc, sem0, sem1, sem2):
    cid = lax.axis_index("c")
    sid = lax.axis_index("s")
    wid = cid * 16 + sid
    pltpu.sync_copy(zeros_hbm, acc.at[pl.ds(sid * ROWS_PER_TILE, ROWS_PER_TILE)])
    plsc.subcore_barrier()

    rows = (rows0, rows1, rows2)
    sems = (sem0, sem1, sem2)

    def chunk(ch, carry):
        pltpu.sync_copy(ei_hbm.at[0, wid, ch], src_v)
        pltpu.sync_copy(ei_hbm.at[1, wid, ch], dst_v)
        pltpu.async_copy(hs_hbm.at[src_v.at[0]], rows0, sem0)
        pltpu.async_copy(hs_hbm.at[src_v.at[1]], rows1, sem1)

        def body(j, c2):
            for k in range(3):
                b = 3 * j + k
                kn = (k + 2) % 3
                pltpu.async_copy(hs_hbm.at[src_v.at[b + 2]], rows[kn], sems[kn])
                pltpu.make_async_copy(
                    hs_hbm.at[src_v.at[b]], rows[k], sems[k]).wait()
                pltpu.sync_copy(rows[k], acc.at[dst_v.at[b]], add=True)
            return c2

        lax.fori_loop(0, (IC - 2) // 3, body, 0)
        for b in range(3 * ((IC - 2) // 3), IC):
            k = b % 3
            if b + 2 < IC:
                kn = (b + 2) % 3
                pltpu.async_copy(hs_hbm.at[src_v.at[b + 2]], rows[kn], sems[kn])
            pltpu.make_async_copy(
                hs_hbm.at[src_v.at[b]], rows[k], sems[k]).wait()
            pltpu.sync_copy(rows[k], acc.at[dst_v.at[b]], add=True)
        return carry

    lax.fori_loop(0, NB // IC, chunk, 0)
    plsc.subcore_barrier()
    pltpu.sync_copy(
        acc.at[pl.ds(sid * ROWS_PER_TILE, ROWS_PER_TILE)],
        out_hbm.at[cid, pl.ds(sid * ROWS_PER_TILE, ROWS_PER_TILE)],
    )


def _deg_body(dst_ref, c_ref):
    i = pl.program_id(0)

    @pl.when(i == 0)
    def _():
        c_ref[...] = jnp.zeros_like(c_ref)

    d = dst_ref[0, 0]
    q = lax.shift_right_logical(d, 7)
    r = lax.bitwise_and(d, 127)
    kq = lax.broadcasted_iota(jnp.int32, (NQ, EB), 0)
    k = lax.broadcasted_iota(jnp.int32, (128, EB), 0)
    m2t = (q == kq).astype(jnp.bfloat16)
    m1t = (r == k).astype(jnp.bfloat16)
    c_ref[...] += lax.dot_general(
        m2t, m1t, (((1,), (1,)), ((), ())),
        preferred_element_type=jnp.float32)


def _deg_tc(dst3):
    return pl.pallas_call(
        _deg_body,
        grid=(NEB,),
        in_specs=[pl.BlockSpec((1, 1, 1, EB), lambda i: (1, i, 0, 0))],
        out_specs=pl.BlockSpec((NQ, 128), lambda i: (0, 0)),
        out_shape=jax.ShapeDtypeStruct((NQ, 128), jnp.float32),
    )(dst3)


def _pick_s(cnt_ref, blk, i):
    n = lax.broadcasted_iota(jnp.int32, (blk, 1), 0) + i * blk
    q = lax.shift_right_logical(n, 7)
    r = lax.bitwise_and(n, 127)
    lane = lax.broadcasted_iota(jnp.int32, (blk, 128), 1)
    laneq = lax.broadcasted_iota(jnp.int32, (blk, NQ), 1)
    eq = (q == laneq).astype(jnp.bfloat16)
    picked = lax.dot_general(
        eq, cnt_ref[...].astype(jnp.bfloat16), (((1,), (0,)), ((), ())),
        preferred_element_type=jnp.float32)
    cnt = jnp.sum(jnp.where(r == lane, picked, 0.0), axis=1, keepdims=True)
    return lax.rsqrt(cnt + 1.0)


def _lin_body(x_ref, w_ref, cnt_ref, hs_ref, s_ref):
    blk = x_ref.shape[0]
    s2 = _pick_s(cnt_ref, blk, pl.program_id(0))
    h = jnp.dot(x_ref[...], w_ref[...], preferred_element_type=jnp.float32)
    hs_ref[...] = h * s2
    s_ref[...] = s2


def _lin_tc(x, W, degc):
    blk = 1000
    return pl.pallas_call(
        _lin_body,
        grid=(N_NODES // blk,),
        in_specs=[
            pl.BlockSpec((blk, D), lambda i: (i, 0)),
            pl.BlockSpec((D, D), lambda i: (0, 0)),
            pl.BlockSpec((NQ, 128), lambda i: (0, 0)),
        ],
        out_specs=[
            pl.BlockSpec((blk, D), lambda i: (i, 0)),
            pl.BlockSpec((blk, 1), lambda i: (i, 0)),
        ],
        out_shape=[
            jax.ShapeDtypeStruct((N_NODES, D), jnp.float32),
            jax.ShapeDtypeStruct((N_NODES, 1), jnp.float32),
        ],
    )(x, W, degc)


def _fin_body(p_ref, hs_ref, s_ref, b_ref, out_ref):
    tot = p_ref[0] + p_ref[1] + hs_ref[...]
    out_ref[...] = jnp.maximum(s_ref[...] * tot + b_ref[...], 0.0)


def _fin_tc(parts, hs, s2d, b2):
    blk = 1000
    return pl.pallas_call(
        _fin_body,
        grid=(N_NODES // blk,),
        in_specs=[
            pl.BlockSpec((2, blk, D), lambda i: (0, i, 0)),
            pl.BlockSpec((blk, D), lambda i: (i, 0)),
            pl.BlockSpec((blk, 1), lambda i: (i, 0)),
            pl.BlockSpec((1, D), lambda i: (0, 0)),
        ],
        out_specs=pl.BlockSpec((blk, D), lambda i: (i, 0)),
        out_shape=jax.ShapeDtypeStruct((N_NODES, D), jnp.float32),
    )(parts, hs, s2d, b2)


def kernel(x, edge_index, W, b):
    ei = edge_index.astype(jnp.int32)
    ei5 = ei.reshape(2, NW, NB // IC, IC, B)
    eid = ei.reshape(2, NEB, 1, EB)
    zeros_row = jnp.zeros((ROWS_PER_TILE, D), jnp.float32)

    degc = _deg_tc(eid)
    hs, s2d = _lin_tc(x, W, degc)
    parts = _agg_kernel(hs, ei5, zeros_row)
    return _fin_tc(parts, hs, s2d, b[None, :])
